# Initial kernel scaffold; baseline (speedup 1.0000x reference)
import jax
import jax.numpy as jnp
from jax.experimental import pallas as pl


def kernel(edge_rep, cycle_rep, e2c_idx, cycle_id, c2e_idx, Wc1, bc1, gc1, bec1, Wc2, bc2, gc2, bec2, Wc3, bc3, We1, be1, ge1, bee1, We2, be2):
    raise NotImplementedError("write your pallas kernel here")


# speedup vs baseline: 1.7700x; 1.7700x over previous
import functools

import jax
import jax.numpy as jnp
from jax import lax
from jax.experimental import pallas as pl
from jax.experimental.pallas import tpu as pltpu
from jax.experimental.pallas import tpu_sc as plsc

_EPS = 1e-5
_SC_CORES = 2
_SC_SUBCORES = 16
_NW = _SC_CORES * _SC_SUBCORES
_CH = 800
_RSEG = 320
_RMM = 4000


def _sc_gather(table, idx):
    n_rows = idx.shape[0]
    d = table.shape[1]
    n_chunks = n_rows // _CH
    nloop = -(-n_chunks // _NW)
    mesh = plsc.VectorSubcoreMesh(core_axis_name="c", subcore_axis_name="s")

    @functools.partial(
        pl.kernel,
        out_type=jax.ShapeDtypeStruct((n_rows, d), jnp.float32),
        mesh=mesh,
        scratch_types=[
            pltpu.VMEM((_CH,), jnp.int32),
            pltpu.VMEM((_CH, d), jnp.float32),
            pltpu.SemaphoreType.DMA,
        ],
    )
    def gk(table_hbm, idx_hbm, out_hbm, idx_v, rows_v, sem):
        wid = lax.axis_index("s") * _SC_CORES + lax.axis_index("c")

        def body(j, carry):
            c = wid + j * _NW

            @pl.when(c < n_chunks)
            def _():
                base = c * _CH
                pltpu.sync_copy(idx_hbm.at[pl.ds(base, _CH)], idx_v)
                pltpu.async_copy(table_hbm.at[idx_v], rows_v, sem).wait()
                pltpu.sync_copy(rows_v, out_hbm.at[pl.ds(base, _CH)])

            return carry

        lax.fori_loop(0, nloop, body, 0)

    return gk(table, idx)


def _fwd_carry(e2c, ids_row, T, R, d):
    def body(e2c_ref, ids_ref, c_ref, prevl_ref, open_ref):
        t = pl.program_id(0)
        ids = ids_ref[0]
        x = e2c_ref[...]
        f = ids[0, 0]
        l = ids[0, R - 1]
        maskl = (ids == l).astype(jnp.float32)
        tail = jnp.dot(maskl, x, preferred_element_type=jnp.float32)
        match = jnp.logical_and(t > 0, prevl_ref[0] == f)
        incoming = jnp.where(match, open_ref[...], 0.0)
        c_ref[0] = incoming
        open_ref[...] = tail + jnp.where(f == l, incoming, 0.0)
        prevl_ref[0] = l

    return pl.pallas_call(
        body,
        grid=(T,),
        in_specs=[
            pl.BlockSpec((R, d), lambda t: (t, 0)),
            pl.BlockSpec((1, 1, R), lambda t: (t, 0, 0)),
        ],
        out_specs=pl.BlockSpec((1, 1, d), lambda t: (t, 0, 0)),
        out_shape=jax.ShapeDtypeStruct((T, 1, d), jnp.float32),
        scratch_shapes=[
            pltpu.SMEM((1,), jnp.int32),
            pltpu.VMEM((1, d), jnp.float32),
        ],
    )(e2c, ids_row)


def _x1_stage(cycle_rep, e2c, ids_row, carr, w1, b1, T, R, d, hdim):
    crtot = T * R

    def body(cyc_ref, e2c_ref, ids_ref, c_ref, w_ref, b_ref,
             x1_ref, st_ref, pf_ref, openb_ref, s1_ref, s2_ref):
        s = pl.program_id(0)
        ids = ids_ref[0]
        x = e2c_ref[...]
        f = ids[0, 0]
        l = ids[0, R - 1]
        idsf = ids.astype(jnp.float32)
        a = jnp.broadcast_to(idsf, (R, R))
        m = (a == a.T).astype(jnp.float32)
        local = jnp.dot(m, x, preferred_element_type=jnp.float32)
        head = jnp.dot(m[0:1, :], x, preferred_element_type=jnp.float32)
        dv = jnp.where(jnp.logical_and(s > 0, pf_ref[0] == l),
                       openb_ref[...], 0.0)
        cv = c_ref[0]
        bcast = local + m[:, 0:1] * cv + m[:, R - 1:R] * dv
        hcat = jnp.concatenate([cyc_ref[...], x, bcast], axis=1)
        x1 = jnp.dot(hcat, w_ref[...],
                     preferred_element_type=jnp.float32) + b_ref[...]
        x1_ref[...] = x1
        ps1 = jnp.sum(x1, axis=0, keepdims=True)
        ps2 = jnp.sum(x1 * x1, axis=0, keepdims=True)

        @pl.when(s == 0)
        def _():
            s1_ref[...] = ps1
            s2_ref[...] = ps2

        @pl.when(s > 0)
        def _():
            s1_ref[...] += ps1
            s2_ref[...] += ps2

        openb_ref[...] = head + jnp.where(f == l, dv, 0.0)
        pf_ref[0] = f

        @pl.when(s == T - 1)
        def _():
            st_ref[...] = jnp.concatenate(
                [s1_ref[...], s2_ref[...],
                 jnp.zeros((6, hdim), jnp.float32)], axis=0)

    rev = lambda s: (T - 1 - s, 0)
    rev3 = lambda s: (T - 1 - s, 0, 0)
    return pl.pallas_call(
        body,
        grid=(T,),
        in_specs=[
            pl.BlockSpec((R, d), rev),
            pl.BlockSpec((R, d), rev),
            pl.BlockSpec((1, 1, R), rev3),
            pl.BlockSpec((1, 1, d), rev3),
            pl.BlockSpec((3 * d, hdim), lambda s: (0, 0)),
            pl.BlockSpec((1, hdim), lambda s: (0, 0)),
        ],
        out_specs=[
            pl.BlockSpec((R, hdim), rev),
            pl.BlockSpec((8, hdim), lambda s: (0, 0)),
        ],
        out_shape=[
            jax.ShapeDtypeStruct((crtot, hdim), jnp.float32),
            jax.ShapeDtypeStruct((8, hdim), jnp.float32),
        ],
        scratch_shapes=[
            pltpu.SMEM((1,), jnp.int32),
            pltpu.VMEM((1, d), jnp.float32),
            pltpu.VMEM((1, hdim), jnp.float32),
            pltpu.VMEM((1, hdim), jnp.float32),
        ],
    )(cycle_rep, e2c, ids_row, carr, w1, b1)


def _bn_mm(x, stats, g, b, w, bias, denom, want_stats):
    n, din = x.shape
    dout = w.shape[1]
    T = n // _RMM
    inv = 1.0 / denom

    def body(x_ref, st_ref, g_ref, b_ref, w_ref, bias_ref, out_ref, *rest):
        s = pl.program_id(0)
        mu = st_ref[0:1, :] * inv
        ex2 = st_ref[1:2, :] * inv
        rstd = lax.rsqrt(jnp.maximum(ex2 - mu * mu, 0.0) + _EPS)
        y = jnp.maximum((x_ref[...] - mu) * rstd * g_ref[...] + b_ref[...],
                        0.0)
        o = jnp.dot(y, w_ref[...],
                    preferred_element_type=jnp.float32) + bias_ref[...]
        out_ref[...] = o
        if want_stats:
            stout_ref, s1_ref, s2_ref = rest
            ps1 = jnp.sum(o, axis=0, keepdims=True)
            ps2 = jnp.sum(o * o, axis=0, keepdims=True)

            @pl.when(s == 0)
            def _():
                s1_ref[...] = ps1
                s2_ref[...] = ps2

            @pl.when(s > 0)
            def _():
                s1_ref[...] += ps1
                s2_ref[...] += ps2

            @pl.when(s == T - 1)
            def _():
                stout_ref[...] = jnp.concatenate(
                    [s1_ref[...], s2_ref[...],
                     jnp.zeros((6, dout), jnp.float32)], axis=0)

    out_shape = [jax.ShapeDtypeStruct((n, dout), jnp.float32)]
    out_specs = [pl.BlockSpec((_RMM, dout), lambda s: (s, 0))]
    scratch = []
    if want_stats:
        out_shape.append(jax.ShapeDtypeStruct((8, dout), jnp.float32))
        out_specs.append(pl.BlockSpec((8, dout), lambda s: (0, 0)))
        scratch = [pltpu.VMEM((1, dout), jnp.float32),
                   pltpu.VMEM((1, dout), jnp.float32)]
    res = pl.pallas_call(
        body,
        grid=(T,),
        in_specs=[
            pl.BlockSpec((_RMM, din), lambda s: (s, 0)),
            pl.BlockSpec((8, din), lambda s: (0, 0)),
            pl.BlockSpec((1, din), lambda s: (0, 0)),
            pl.BlockSpec((1, din), lambda s: (0, 0)),
            pl.BlockSpec((din, dout), lambda s: (0, 0)),
            pl.BlockSpec((1, dout), lambda s: (0, 0)),
        ],
        out_specs=out_specs,
        out_shape=out_shape,
        scratch_shapes=scratch,
    )(x, stats, g, b, w, bias)
    return res if want_stats else res[0]


def _cat_mm(a, bgat, w, bias):
    n, d = a.shape
    dout = w.shape[1]
    T = n // _RMM

    def body(a_ref, bg_ref, w_ref, bias_ref, out_ref, stout_ref,
             s1_ref, s2_ref):
        s = pl.program_id(0)
        hcat = jnp.concatenate([a_ref[...], bg_ref[...]], axis=1)
        o = jnp.dot(hcat, w_ref[...],
                    preferred_element_type=jnp.float32) + bias_ref[...]
        out_ref[...] = o
        ps1 = jnp.sum(o, axis=0, keepdims=True)
        ps2 = jnp.sum(o * o, axis=0, keepdims=True)

        @pl.when(s == 0)
        def _():
            s1_ref[...] = ps1
            s2_ref[...] = ps2

        @pl.when(s > 0)
        def _():
            s1_ref[...] += ps1
            s2_ref[...] += ps2

        @pl.when(s == T - 1)
        def _():
            stout_ref[...] = jnp.concatenate(
                [s1_ref[...], s2_ref[...],
                 jnp.zeros((6, dout), jnp.float32)], axis=0)

    return pl.pallas_call(
        body,
        grid=(T,),
        in_specs=[
            pl.BlockSpec((_RMM, d), lambda s: (s, 0)),
            pl.BlockSpec((_RMM, d), lambda s: (s, 0)),
            pl.BlockSpec((2 * d, dout), lambda s: (0, 0)),
            pl.BlockSpec((1, dout), lambda s: (0, 0)),
        ],
        out_specs=[
            pl.BlockSpec((_RMM, dout), lambda s: (s, 0)),
            pl.BlockSpec((8, dout), lambda s: (0, 0)),
        ],
        out_shape=[
            jax.ShapeDtypeStruct((n, dout), jnp.float32),
            jax.ShapeDtypeStruct((8, dout), jnp.float32),
        ],
        scratch_shapes=[pltpu.VMEM((1, dout), jnp.float32),
                        pltpu.VMEM((1, dout), jnp.float32)],
    )(a, bgat, w, bias)


def kernel(edge_rep, cycle_rep, e2c_idx, cycle_id, c2e_idx,
           Wc1, bc1, gc1, bec1, Wc2, bc2, gc2, bec2, Wc3, bc3,
           We1, be1, ge1, bee1, We2, be2):
    e, d = edge_rep.shape
    cr = cycle_rep.shape[0]
    hdim = Wc1.shape[1]
    T = cr // _RSEG

    ids_row = cycle_id.astype(jnp.int32).reshape(T, 1, _RSEG)

    e2c = _sc_gather(edge_rep, e2c_idx.astype(jnp.int32))
    carr = _fwd_carry(e2c, ids_row, T, _RSEG, d)
    x1, st1 = _x1_stage(cycle_rep, e2c, ids_row, carr,
                        Wc1, bc1.reshape(1, hdim), T, _RSEG, d, hdim)
    x2, st2 = _bn_mm(x1, st1, gc1.reshape(1, hdim), bec1.reshape(1, hdim),
                     Wc2, bc2.reshape(1, hdim), cr, True)
    cycle_out = _bn_mm(x2, st2, gc2.reshape(1, hdim), bec2.reshape(1, hdim),
                       Wc3, bc3.reshape(1, d), cr, False)
    c2e = _sc_gather(cycle_out, c2e_idx.astype(jnp.int32))
    x3, st3 = _cat_mm(edge_rep, c2e, We1, be1.reshape(1, hdim))
    edge_out = _bn_mm(x3, st3, ge1.reshape(1, hdim), bee1.reshape(1, hdim),
                      We2, be2.reshape(1, d), e, False)
    return (edge_out, cycle_out)
